# Initial kernel scaffold; baseline (speedup 1.0000x reference)
import jax
import jax.numpy as jnp
from jax.experimental import pallas as pl


def kernel(x, edge_index, W0, b0, W1, b1, W2, b2, W3, b3, Wp1, bp1, Wp2, bp2):
    raise NotImplementedError("write your pallas kernel here")


# speedup vs baseline: 7.4067x; 7.4067x over previous
import functools

import jax
import jax.numpy as jnp
from jax import lax
from jax.experimental import pallas as pl
from jax.experimental.pallas import tpu as pltpu
from jax.experimental.pallas import tpu_sc as plsc

N = 10000
D = 128
D_OUT = 64
E = 320000

NC = 2
NS = 16
NW = NC * NS
CH = 128
G = 79
EPT = G * CH
E_PAD = NW * EPT
N_PAD = 10112
STRIPE = N_PAD // NS
BM = 1000

_MESH = plsc.VectorSubcoreMesh(core_axis_name="c", subcore_axis_name="s")


@functools.partial(
    pl.kernel,
    out_type=jax.ShapeDtypeStruct((NC, N_PAD, 16), jnp.float32),
    mesh=_MESH,
    scratch_types=[
        pltpu.VMEM((G, CH), jnp.int32),
        pltpu.VMEM((CH, 16), jnp.float32),
        pltpu.VMEM_SHARED((N_PAD, 16), jnp.float32),
    ],
)
def _deg_sc(dst_hbm, ones_hbm, zeros_hbm, out_hbm, didx_v, ones_v, deg_sh):
    cid = lax.axis_index("c")
    sid = lax.axis_index("s")
    wid = sid * NC + cid
    pltpu.sync_copy(dst_hbm.at[wid], didx_v)
    pltpu.sync_copy(ones_hbm, ones_v)
    r0 = sid * STRIPE
    pltpu.sync_copy(zeros_hbm.at[pl.ds(r0, STRIPE)], deg_sh.at[pl.ds(r0, STRIPE)])
    plsc.subcore_barrier()

    def step(g, carry):
        pltpu.sync_copy(ones_v, deg_sh.at[didx_v.at[g]], add=True)
        return carry

    lax.fori_loop(0, G, step, 0)
    plsc.subcore_barrier()
    pltpu.sync_copy(deg_sh.at[pl.ds(r0, STRIPE)],
                    out_hbm.at[cid].at[pl.ds(r0, STRIPE)])


@functools.partial(
    pl.kernel,
    out_type=jax.ShapeDtypeStruct((NC, N_PAD, D), jnp.float32),
    mesh=_MESH,
    scratch_types=[
        pltpu.VMEM((G, CH), jnp.int32),
        pltpu.VMEM((G, CH), jnp.int32),
        pltpu.VMEM((CH, D), jnp.float32),
        pltpu.VMEM_SHARED((N_PAD, D), jnp.float32),
        pltpu.SemaphoreType.DMA,
    ],
)
def _agg_sc(src_hbm, dst_hbm, table_hbm, zeros_hbm, out_hbm,
            sidx_v, didx_v, rows_v, acc_sh, sem):
    cid = lax.axis_index("c")
    sid = lax.axis_index("s")
    wid = sid * NC + cid
    pltpu.sync_copy(src_hbm.at[wid], sidx_v)
    pltpu.sync_copy(dst_hbm.at[wid], didx_v)
    r0 = sid * STRIPE
    pltpu.sync_copy(zeros_hbm.at[pl.ds(r0, STRIPE)], acc_sh.at[pl.ds(r0, STRIPE)])
    plsc.subcore_barrier()

    def step(g, carry):
        pltpu.async_copy(table_hbm.at[sidx_v.at[g]], rows_v, sem).wait()
        pltpu.sync_copy(rows_v, acc_sh.at[didx_v.at[g]], add=True)
        return carry

    lax.fori_loop(0, G, step, 0)
    plsc.subcore_barrier()
    pltpu.sync_copy(acc_sh.at[pl.ds(r0, STRIPE)],
                    out_hbm.at[cid].at[pl.ds(r0, STRIPE)])


def _k_in_body(x_ref, w_ref, deg_ref, hs_ref, dinv_ref):
    dinv = lax.rsqrt(deg_ref[0] + deg_ref[1] + 1.0)
    h = jnp.dot(x_ref[...], w_ref[...], preferred_element_type=jnp.float32)
    hs_ref[...] = h * dinv
    dinv_ref[...] = dinv


def _k_in(x, w, degs):
    return pl.pallas_call(
        _k_in_body,
        grid=(N // BM,),
        in_specs=[
            pl.BlockSpec((BM, D), lambda i: (i, 0)),
            pl.BlockSpec((D, D), lambda i: (0, 0)),
            pl.BlockSpec((NC, BM, 1), lambda i: (0, i, 0)),
        ],
        out_specs=[
            pl.BlockSpec((BM, D), lambda i: (i, 0)),
            pl.BlockSpec((BM, 1), lambda i: (i, 0)),
        ],
        out_shape=[
            jax.ShapeDtypeStruct((N, D), jnp.float32),
            jax.ShapeDtypeStruct((N, 1), jnp.float32),
        ],
    )(x, w, degs)


def _k_layer_body(acc_ref, hs_ref, dinv_ref, b_ref, w_ref, out_ref):
    comb = acc_ref[0] + acc_ref[1] + hs_ref[...]
    xnew = jnp.maximum(dinv_ref[...] * comb + b_ref[...], 0.0)
    out_ref[...] = jnp.dot(xnew, w_ref[...],
                           preferred_element_type=jnp.float32) * dinv_ref[...]


def _k_layer(acc, hs, dinv, b, w):
    return pl.pallas_call(
        _k_layer_body,
        grid=(N // BM,),
        in_specs=[
            pl.BlockSpec((NC, BM, D), lambda i: (0, i, 0)),
            pl.BlockSpec((BM, D), lambda i: (i, 0)),
            pl.BlockSpec((BM, 1), lambda i: (i, 0)),
            pl.BlockSpec((1, D), lambda i: (0, 0)),
            pl.BlockSpec((D, D), lambda i: (0, 0)),
        ],
        out_specs=pl.BlockSpec((BM, D), lambda i: (i, 0)),
        out_shape=jax.ShapeDtypeStruct((N, D), jnp.float32),
    )(acc, hs, dinv, b, w)


def _k_final_body(acc_ref, hs_ref, dinv_ref, b3_ref, wp1_ref, bp1_ref,
                  wp2_ref, bp2_ref, out_ref):
    comb = acc_ref[0] + acc_ref[1] + hs_ref[...]
    x4 = jnp.maximum(dinv_ref[...] * comb + b3_ref[...], 0.0)
    z = jnp.dot(x4, wp1_ref[...], preferred_element_type=jnp.float32) + bp1_ref[...]
    z = jnp.dot(z, wp2_ref[...], preferred_element_type=jnp.float32) + bp2_ref[...]
    m = jnp.max(z, axis=1, keepdims=True)
    ez = jnp.exp(z - m)
    out_ref[...] = z - m - jnp.log(jnp.sum(ez, axis=1, keepdims=True))


def _k_final(acc, hs, dinv, b3, wp1, bp1, wp2, bp2):
    return pl.pallas_call(
        _k_final_body,
        grid=(N // BM,),
        in_specs=[
            pl.BlockSpec((NC, BM, D), lambda i: (0, i, 0)),
            pl.BlockSpec((BM, D), lambda i: (i, 0)),
            pl.BlockSpec((BM, 1), lambda i: (i, 0)),
            pl.BlockSpec((1, D), lambda i: (0, 0)),
            pl.BlockSpec((D, D), lambda i: (0, 0)),
            pl.BlockSpec((1, D), lambda i: (0, 0)),
            pl.BlockSpec((D, D_OUT), lambda i: (0, 0)),
            pl.BlockSpec((1, D_OUT), lambda i: (0, 0)),
        ],
        out_specs=pl.BlockSpec((BM, D_OUT), lambda i: (i, 0)),
        out_shape=jax.ShapeDtypeStruct((N, D_OUT), jnp.float32),
    )(acc, hs, dinv, b3, wp1, bp1, wp2, bp2)


def kernel(x, edge_index, W0, b0, W1, b1, W2, b2, W3, b3, Wp1, bp1, Wp2, bp2):
    src = edge_index[0]
    dst = edge_index[1]
    pad = E_PAD - E
    src_p = jnp.concatenate([src, jnp.zeros((pad,), jnp.int32)]).reshape(NW, G, CH)
    dst_p = jnp.concatenate([dst, jnp.full((pad,), N, jnp.int32)]).reshape(NW, G, CH)

    ones16 = jnp.ones((CH, 16), jnp.float32)
    zeros16 = jnp.zeros((N_PAD, 16), jnp.float32)
    zerosD = jnp.zeros((N_PAD, D), jnp.float32)

    degs = _deg_sc(dst_p, ones16, zeros16)
    degs = degs[:, :N, 0:1]

    hs, dinv = _k_in(x, W0, degs)
    for b_prev, w_next in ((b0, W1), (b1, W2), (b2, W3)):
        acc = _agg_sc(src_p, dst_p, hs, zerosD)
        hs = _k_layer(acc[:, :N, :], hs, dinv, b_prev.reshape(1, D), w_next)
    acc = _agg_sc(src_p, dst_p, hs, zerosD)
    return _k_final(acc[:, :N, :], hs, dinv, b3.reshape(1, D),
                    Wp1, bp1.reshape(1, D), Wp2, bp2.reshape(1, D_OUT))
